# baseline (speedup 1.0000x reference)
import functools

import jax
import jax.numpy as jnp
from jax import lax
from jax.experimental import pallas as pl
from jax.experimental.pallas import tpu as pltpu
from jax.experimental.pallas import tpu_sc as plsc

BATCH = 16384
TIMESTEPS = 200
EMBED_DIM = 32

NC = 2
NS = 16
NW = NC * NS

BT = 128
U = 4
COLS = U * BT
NBUF = 3
NGRP = TIMESTEPS
DT = EMBED_DIM // 8


def _make_kernel():
    mesh = plsc.VectorSubcoreMesh(core_axis_name="c", subcore_axis_name="s")

    @functools.partial(
        pl.kernel,
        mesh=mesh,
        out_type=jax.ShapeDtypeStruct(
            (TIMESTEPS, DT, BATCH // BT, 8, BT), jnp.float32),
        scratch_types=[
            pltpu.VMEM((NBUF, COLS), jnp.int32),
            pltpu.VMEM((NBUF, U, BT, EMBED_DIM), jnp.float32),
            pltpu.VMEM((NBUF, U, EMBED_DIM, BT), jnp.float32),
        ] + [pltpu.SemaphoreType.DMA] * (2 * NBUF),
        compiler_params=pltpu.CompilerParams(
            use_tc_tiling_on_sc=False, needs_layout_passes=False),
    )
    def emb_kernel(itemsT_hbm, table_hbm, out_hbm, idx_v, raw_v, trans_v,
                   *sems):
        gsem = sems[:NBUF]
        osem = sems[NBUF:]
        wid = lax.axis_index("s") * NC + lax.axis_index("c")
        col0 = wid * COLS
        bt0 = wid * U
        iota16 = lax.iota(jnp.int32, 16)
        bvecs = [iota16 + k * 16 for k in range(BT // 16)]

        def stage_and_fire(t, b):
            pltpu.sync_copy(itemsT_hbm.at[t, pl.ds(col0, COLS)], idx_v.at[b])
            for j in range(U):
                pltpu.async_copy(
                    table_hbm.at[idx_v.at[b, pl.ds(j * BT, BT)]],
                    raw_v.at[b, j], gsem[b])

        def wait_gathers(b):
            for j in range(U):
                pltpu.make_async_copy(
                    table_hbm.at[idx_v.at[b, pl.ds(j * BT, BT)]],
                    raw_v.at[b, j], gsem[b]).wait()

        def transpose(b):
            bsplat = iota16 * 0 + b

            def body_i(i, carry):
                j = i >> 3
                jvec = iota16 * 0 + j
                for dd in range(4):
                    d = (i & 7) * 4 + dd
                    dvec = iota16 * 0 + d
                    for k in range(BT // 16):
                        vals = plsc.load_gather(
                            raw_v, [bsplat, jvec, bvecs[k], dvec])
                        trans_v[b, j, d, pl.ds(k * 16, 16)] = vals
                return carry

            lax.fori_loop(0, U * 8, body_i, 0)

        def fire_stores(t, b):
            for j in range(U):
                for dt in range(DT):
                    pltpu.async_copy(
                        trans_v.at[b, j, pl.ds(dt * 8, 8)],
                        out_hbm.at[t, dt, bt0 + j], osem[b])

        def wait_stores(t, b):
            for j in range(U):
                for dt in range(DT):
                    pltpu.make_async_copy(
                        trans_v.at[b, j, pl.ds(dt * 8, 8)],
                        out_hbm.at[t, dt, bt0 + j], osem[b]).wait()

        def complete(t, b):
            wait_gathers(b)
            transpose(b)
            fire_stores(t, b)

        for g0 in range(NBUF):
            stage_and_fire(g0, g0)
            if g0 >= 1:
                complete(g0 - 1, g0 - 1)

        nk = (NGRP - 1) // NBUF + 1

        def body(k, carry):
            for off in range(NBUF):
                b1 = (off + NBUF - 1) % NBUF

                def work(off=off, b1=b1):
                    g = k * NBUF + off
                    wait_stores(g - NBUF, off)
                    stage_and_fire(g, off)
                    complete(g - 1, b1)

                if (nk - 1) * NBUF + off >= NGRP:
                    pl.when(k < nk - 1)(work)
                else:
                    work()
            return carry

        lax.fori_loop(1, nk, body, 0)

        complete(NGRP - 1, (NGRP - 1) % NBUF)
        for g0 in range(NGRP - NBUF, NGRP):
            wait_stores(g0, g0 % NBUF)

    return emb_kernel


_EMB = _make_kernel()


def kernel(items, table):
    itemsT = jnp.transpose(items).astype(jnp.int32)
    out5 = _EMB(itemsT, table)
    return out5.transpose(2, 4, 0, 1, 3).reshape(BATCH, TIMESTEPS, EMBED_DIM)


# speedup vs baseline: 2.2293x; 2.2293x over previous
import functools

import jax
import jax.numpy as jnp
from jax import lax
from jax.experimental import pallas as pl
from jax.experimental.pallas import tpu as pltpu
from jax.experimental.pallas import tpu_sc as plsc

BATCH = 16384
TIMESTEPS = 200
EMBED_DIM = 32
PADD = EMBED_DIM + 8

NC = 2
NS = 16
NW = NC * NS

BT = 128
U = 4
COLS = U * BT
NBUF = 3
NGRP = TIMESTEPS
DT = EMBED_DIM // 8


def _make_kernel():
    mesh = plsc.VectorSubcoreMesh(core_axis_name="c", subcore_axis_name="s")

    @functools.partial(
        pl.kernel,
        mesh=mesh,
        out_type=jax.ShapeDtypeStruct(
            (TIMESTEPS, DT, BATCH // BT, 8, BT), jnp.float32),
        scratch_types=[
            pltpu.VMEM((NBUF, COLS), jnp.int32),
            pltpu.VMEM((NBUF, U, BT, PADD), jnp.float32),
            pltpu.VMEM((NBUF, U, EMBED_DIM, BT), jnp.float32),
        ] + [pltpu.SemaphoreType.DMA] * (3 * NBUF),
        compiler_params=pltpu.CompilerParams(
            use_tc_tiling_on_sc=False, needs_layout_passes=False),
    )
    def emb_kernel(itemsT_hbm, table_hbm, out_hbm, idx_v, raw_v, trans_v,
                   *sems):
        gsem = sems[:NBUF]
        osem = sems[NBUF:2 * NBUF]
        isem = sems[2 * NBUF:]
        wid = lax.axis_index("s") * NC + lax.axis_index("c")
        col0 = wid * COLS
        bt0 = wid * U
        iota16 = lax.iota(jnp.int32, 16)
        bvecs = [iota16 + k * 16 for k in range(BT // 16)]

        def fetch_idx(t, b):
            pltpu.async_copy(
                itemsT_hbm.at[t, pl.ds(col0, COLS)], idx_v.at[b], isem[b])

        def wait_idx(t, b):
            pltpu.make_async_copy(
                itemsT_hbm.at[t, pl.ds(col0, COLS)], idx_v.at[b], isem[b]
            ).wait()

        def fire_gathers(b):
            for j in range(U):
                pltpu.async_copy(
                    table_hbm.at[idx_v.at[b, pl.ds(j * BT, BT)]],
                    raw_v.at[b, j], gsem[b])

        def wait_gathers(b):
            for j in range(U):
                pltpu.make_async_copy(
                    table_hbm.at[idx_v.at[b, pl.ds(j * BT, BT)]],
                    raw_v.at[b, j], gsem[b]).wait()

        def transpose(b):
            bsplat = iota16 * 0 + b

            def body_i(i, carry):
                j = i >> 3
                jvec = iota16 * 0 + j
                for dd in range(4):
                    d = (i & 7) * 4 + dd
                    dvec = iota16 * 0 + d
                    for k in range(BT // 16):
                        vals = plsc.load_gather(
                            raw_v, [bsplat, jvec, bvecs[k], dvec])
                        trans_v[b, j, d, pl.ds(k * 16, 16)] = vals
                return carry

            lax.fori_loop(0, U * 8, body_i, 0)

        def fire_stores(t, b):
            for j in range(U):
                for dt in range(DT):
                    pltpu.async_copy(
                        trans_v.at[b, j, pl.ds(dt * 8, 8)],
                        out_hbm.at[t, dt, bt0 + j], osem[b])

        def wait_stores(t, b):
            for j in range(U):
                for dt in range(DT):
                    pltpu.make_async_copy(
                        trans_v.at[b, j, pl.ds(dt * 8, 8)],
                        out_hbm.at[t, dt, bt0 + j], osem[b]).wait()

        def complete(t, b):
            wait_gathers(b)
            transpose(b)
            fire_stores(t, b)

        fetch_idx(0, 0)
        wait_idx(0, 0)
        fire_gathers(0)
        fetch_idx(1, 1)
        wait_idx(1, 1)
        fire_gathers(1)
        fetch_idx(2, 2)
        complete(0, 0)
        wait_idx(2, 2)
        fire_gathers(2)
        fetch_idx(3, 0)
        complete(1, 1)

        nk = (NGRP - 1) // NBUF + 1

        def body(k, carry):
            for off in range(NBUF):
                b1 = (off + NBUF - 1) % NBUF
                bn = (off + 1) % NBUF

                def work(off=off, b1=b1, bn=bn):
                    g = k * NBUF + off
                    wait_stores(g - NBUF, off)
                    wait_idx(g, off)
                    fire_gathers(off)
                    if (nk - 1) * NBUF + off + 1 >= NGRP:
                        pl.when(k < nk - 1)(lambda: fetch_idx(g + 1, bn))
                    else:
                        fetch_idx(g + 1, bn)
                    complete(g - 1, b1)

                if (nk - 1) * NBUF + off >= NGRP:
                    pl.when(k < nk - 1)(work)
                else:
                    work()
            return carry

        lax.fori_loop(1, nk, body, 0)

        complete(NGRP - 1, (NGRP - 1) % NBUF)
        for g0 in range(NGRP - NBUF, NGRP):
            wait_stores(g0, g0 % NBUF)

    return emb_kernel


_EMB = _make_kernel()


def kernel(items, table):
    itemsT = jnp.transpose(items).astype(jnp.int32)
    table_p = jnp.pad(table, ((0, 0), (0, PADD - EMBED_DIM)))
    out5 = _EMB(itemsT, table_p)
    return out5.transpose(2, 4, 0, 1, 3).reshape(BATCH, TIMESTEPS, EMBED_DIM)


# speedup vs baseline: 2.7874x; 1.2504x over previous
import functools

import jax
import jax.numpy as jnp
from jax import lax
from jax.experimental import pallas as pl
from jax.experimental.pallas import tpu as pltpu
from jax.experimental.pallas import tpu_sc as plsc

BATCH = 16384
TIMESTEPS = 200
EMBED_DIM = 32
PADD = EMBED_DIM + 8

NC = 2
NS = 16
NW = NC * NS

BT = 128
U = 4
COLS = U * BT
NBUF = 3
NGRP = TIMESTEPS
DT = EMBED_DIM // 8


def _make_kernel():
    mesh = plsc.VectorSubcoreMesh(core_axis_name="c", subcore_axis_name="s")

    @functools.partial(
        pl.kernel,
        mesh=mesh,
        out_type=jax.ShapeDtypeStruct(
            (TIMESTEPS, DT, BATCH // BT, 8, BT), jnp.float32),
        scratch_types=[
            pltpu.VMEM((NBUF, COLS), jnp.int32),
            pltpu.VMEM((NBUF, U, BT, PADD), jnp.float32),
            pltpu.VMEM((NBUF, U, EMBED_DIM, BT), jnp.float32),
        ] + [pltpu.SemaphoreType.DMA] * (3 * NBUF),
        compiler_params=pltpu.CompilerParams(
            use_tc_tiling_on_sc=False, needs_layout_passes=False),
    )
    def emb_kernel(itemsT_hbm, table_hbm, out_hbm, idx_v, raw_v, trans_v,
                   *sems):
        gsem = sems[:NBUF]
        osem = sems[NBUF:2 * NBUF]
        isem = sems[2 * NBUF:]
        wid = lax.axis_index("s") * NC + lax.axis_index("c")
        col0 = wid * COLS
        bt0 = wid * U
        iota16 = lax.iota(jnp.int32, 16)
        bvecs = [iota16 + k * 16 for k in range(BT // 16)]

        def fetch_idx(t, b):
            pltpu.async_copy(
                itemsT_hbm.at[t, pl.ds(col0, COLS)], idx_v.at[b], isem[b])

        def wait_idx(t, b):
            pltpu.make_async_copy(
                itemsT_hbm.at[t, pl.ds(col0, COLS)], idx_v.at[b], isem[b]
            ).wait()

        def fire_gathers(b):
            for j in range(U):
                pltpu.async_copy(
                    table_hbm.at[idx_v.at[b, pl.ds(j * BT, BT)]],
                    raw_v.at[b, j], gsem[b])

        def wait_gathers(b):
            for j in range(U):
                pltpu.make_async_copy(
                    table_hbm.at[idx_v.at[b, pl.ds(j * BT, BT)]],
                    raw_v.at[b, j], gsem[b]).wait()

        def transpose(b):
            bsplat = iota16 * 0 + b

            def body_i(i, carry):
                j = i >> 3
                jvec = iota16 * 0 + j
                for dd in range(4):
                    d0 = (i & 7) * 4 + dd
                    dvec = (iota16 + d0) & (EMBED_DIM - 1)
                    for k in range(BT // 16):
                        vals = plsc.load_gather(
                            raw_v, [bsplat, jvec, bvecs[k], dvec])
                        plsc.store_scatter(
                            trans_v, [bsplat, jvec, dvec, bvecs[k]], vals)
                return carry

            lax.fori_loop(0, U * 8, body_i, 0)

        def fire_stores(t, b):
            for j in range(U):
                for dt in range(DT):
                    pltpu.async_copy(
                        trans_v.at[b, j, pl.ds(dt * 8, 8)],
                        out_hbm.at[t, dt, bt0 + j], osem[b])

        def wait_stores(t, b):
            for j in range(U):
                for dt in range(DT):
                    pltpu.make_async_copy(
                        trans_v.at[b, j, pl.ds(dt * 8, 8)],
                        out_hbm.at[t, dt, bt0 + j], osem[b]).wait()

        def complete(t, b):
            wait_gathers(b)
            transpose(b)
            fire_stores(t, b)

        fetch_idx(0, 0)
        wait_idx(0, 0)
        fire_gathers(0)
        fetch_idx(1, 1)
        wait_idx(1, 1)
        fire_gathers(1)
        fetch_idx(2, 2)
        complete(0, 0)
        wait_idx(2, 2)
        fire_gathers(2)
        fetch_idx(3, 0)
        complete(1, 1)

        nk = (NGRP - 1) // NBUF + 1

        def body(k, carry):
            for off in range(NBUF):
                b1 = (off + NBUF - 1) % NBUF
                bn = (off + 1) % NBUF

                def work(off=off, b1=b1, bn=bn):
                    g = k * NBUF + off
                    wait_stores(g - NBUF, off)
                    wait_idx(g, off)
                    fire_gathers(off)
                    if (nk - 1) * NBUF + off + 1 >= NGRP:
                        pl.when(k < nk - 1)(lambda: fetch_idx(g + 1, bn))
                    else:
                        fetch_idx(g + 1, bn)
                    complete(g - 1, b1)

                if (nk - 1) * NBUF + off >= NGRP:
                    pl.when(k < nk - 1)(work)
                else:
                    work()
            return carry

        lax.fori_loop(1, nk, body, 0)

        complete(NGRP - 1, (NGRP - 1) % NBUF)
        for g0 in range(NGRP - NBUF, NGRP):
            wait_stores(g0, g0 % NBUF)

    return emb_kernel


_EMB = _make_kernel()


def kernel(items, table):
    itemsT = jnp.transpose(items).astype(jnp.int32)
    table_p = jnp.pad(table, ((0, 0), (0, PADD - EMBED_DIM)))
    out5 = _EMB(itemsT, table_p)
    return out5.transpose(2, 4, 0, 1, 3).reshape(BATCH, TIMESTEPS, EMBED_DIM)


# speedup vs baseline: 5.0270x; 1.8035x over previous
import functools

import jax
import jax.numpy as jnp
from jax import lax
from jax.experimental import pallas as pl
from jax.experimental.pallas import tpu as pltpu
from jax.experimental.pallas import tpu_sc as plsc

BATCH = 16384
TIMESTEPS = 200
EMBED_DIM = 32
PADD = EMBED_DIM

NC = 2
NS = 16
NW = NC * NS

BT = 128
U = 4
COLS = U * BT
NBUF = 3
NGRP = TIMESTEPS
DT = EMBED_DIM // 8


def _make_kernel():
    mesh = plsc.VectorSubcoreMesh(core_axis_name="c", subcore_axis_name="s")

    @functools.partial(
        pl.kernel,
        mesh=mesh,
        out_type=jax.ShapeDtypeStruct(
            (TIMESTEPS, DT, BATCH // BT, 8, BT), jnp.float32),
        scratch_types=[
            pltpu.VMEM((NBUF, COLS), jnp.int32),
            pltpu.VMEM((NBUF, U, BT, PADD), jnp.float32),
            pltpu.VMEM((NBUF, U, EMBED_DIM, BT), jnp.float32),
        ] + [pltpu.SemaphoreType.DMA] * (3 * NBUF),
        compiler_params=pltpu.CompilerParams(
            use_tc_tiling_on_sc=False, needs_layout_passes=False),
    )
    def emb_kernel(itemsT_hbm, table_hbm, out_hbm, idx_v, raw_v, trans_v,
                   *sems):
        gsem = sems[:NBUF]
        osem = sems[NBUF:2 * NBUF]
        isem = sems[2 * NBUF:]
        wid = lax.axis_index("s") * NC + lax.axis_index("c")
        col0 = wid * COLS
        bt0 = wid * U
        iota16 = lax.iota(jnp.int32, 16)
        bvecs = [iota16 + k * 16 for k in range(BT // 16)]

        def fetch_idx(t, b):
            pltpu.async_copy(
                itemsT_hbm.at[t, pl.ds(col0, COLS)], idx_v.at[b], isem[b])

        def wait_idx(t, b):
            pltpu.make_async_copy(
                itemsT_hbm.at[t, pl.ds(col0, COLS)], idx_v.at[b], isem[b]
            ).wait()

        def fire_gathers(b):
            for j in range(U):
                pltpu.async_copy(
                    table_hbm.at[idx_v.at[b, pl.ds(j * BT, BT)]],
                    raw_v.at[b, j], gsem[b])

        def wait_gathers(b):
            for j in range(U):
                pltpu.make_async_copy(
                    table_hbm.at[idx_v.at[b, pl.ds(j * BT, BT)]],
                    raw_v.at[b, j], gsem[b]).wait()

        def transpose(b):
            bsplat = iota16 * 0 + b

            @plsc.parallel_loop(0, U * 8, step=1, unroll=2)
            def body_i(i):
                j = i >> 3
                jvec = iota16 * 0 + j
                for dd in range(4):
                    d0 = (i & 7) * 4 + dd
                    dvec = (iota16 + d0) & (EMBED_DIM - 1)
                    for k in range(BT // 16):
                        vals = plsc.load_gather(
                            raw_v, [bsplat, jvec, bvecs[k], dvec])
                        plsc.store_scatter(
                            trans_v, [bsplat, jvec, dvec, bvecs[k]], vals)

        def fire_stores(t, b):
            for j in range(U):
                for dt in range(DT):
                    pltpu.async_copy(
                        trans_v.at[b, j, pl.ds(dt * 8, 8)],
                        out_hbm.at[t, dt, bt0 + j], osem[b])

        def wait_stores(t, b):
            for j in range(U):
                for dt in range(DT):
                    pltpu.make_async_copy(
                        trans_v.at[b, j, pl.ds(dt * 8, 8)],
                        out_hbm.at[t, dt, bt0 + j], osem[b]).wait()

        def complete(t, b):
            wait_gathers(b)
            transpose(b)
            fire_stores(t, b)

        fetch_idx(0, 0)
        wait_idx(0, 0)
        fire_gathers(0)
        fetch_idx(1, 1)
        wait_idx(1, 1)
        fire_gathers(1)
        fetch_idx(2, 2)
        complete(0, 0)
        wait_idx(2, 2)
        fire_gathers(2)
        fetch_idx(3, 0)
        complete(1, 1)

        nk = (NGRP - 1) // NBUF + 1

        def body(k, carry):
            for off in range(NBUF):
                b1 = (off + NBUF - 1) % NBUF
                bn = (off + 1) % NBUF

                def work(off=off, b1=b1, bn=bn):
                    g = k * NBUF + off
                    wait_stores(g - NBUF, off)
                    wait_idx(g, off)
                    fire_gathers(off)
                    if (nk - 1) * NBUF + off + 1 >= NGRP:
                        pl.when(k < nk - 1)(lambda: fetch_idx(g + 1, bn))
                    else:
                        fetch_idx(g + 1, bn)
                    complete(g - 1, b1)

                if (nk - 1) * NBUF + off >= NGRP:
                    pl.when(k < nk - 1)(work)
                else:
                    work()
            return carry

        lax.fori_loop(1, nk, body, 0)

        complete(NGRP - 1, (NGRP - 1) % NBUF)
        for g0 in range(NGRP - NBUF, NGRP):
            wait_stores(g0, g0 % NBUF)

    return emb_kernel


_EMB = _make_kernel()


def kernel(items, table):
    itemsT = jnp.transpose(items).astype(jnp.int32)
    table_p = jnp.pad(table, ((0, 0), (0, PADD - EMBED_DIM)))
    out5 = _EMB(itemsT, table_p)
    return out5.transpose(2, 4, 0, 1, 3).reshape(BATCH, TIMESTEPS, EMBED_DIM)
